# baseline (speedup 1.0000x reference)
import functools

import jax
import jax.numpy as jnp
from jax import lax
from jax.experimental import pallas as pl
from jax.experimental.pallas import tpu as pltpu
from jax.experimental.pallas import tpu_sc as plsc

_NC = 2
_NS = 16
_NW = _NC * _NS


def _sc_body(rows_per_w, batch, h_hbm, lab_hbm, c_hbm, out_hbm,
             idx_v, xv, crow_v, pv, sem, gsem):
    cid = lax.axis_index("c")
    sid = lax.axis_index("s")
    wid = sid * _NC + cid
    base = wid * rows_per_w

    h_cp = pltpu.async_copy(h_hbm.at[pl.ds(base, rows_per_w), :], xv, sem)
    pltpu.sync_copy(lab_hbm, idx_v)
    pltpu.async_copy(c_hbm.at[idx_v], crow_v, gsem).wait()
    h_cp.wait()

    zeros = jnp.zeros((16,), jnp.float32)

    def body(r, carry):
        c0, c1, c2, c3, sq = carry
        v0 = xv[r, pl.ds(0, 16)]
        v1 = xv[r, pl.ds(16, 16)]
        v2 = xv[r, pl.ds(32, 16)]
        v3 = xv[r, pl.ds(48, 16)]
        sq = sq + v0 * v0 + v1 * v1 + v2 * v2 + v3 * v3
        return (c0 + v0, c1 + v1, c2 + v2, c3 + v3, sq)

    c0, c1, c2, c3, sq = lax.fori_loop(
        0, rows_per_w, body, (zeros, zeros, zeros, zeros, zeros)
    )

    r0 = crow_v[0, pl.ds(0, 16)]
    r1 = crow_v[0, pl.ds(16, 16)]
    r2 = crow_v[0, pl.ds(32, 16)]
    r3 = crow_v[0, pl.ds(48, 16)]
    csq_share = jnp.float32(batch / _NW)
    partial = (sq - 2.0 * (c0 * r0 + c1 * r1 + c2 * r2 + c3 * r3)
               + csq_share * (r0 * r0 + r1 * r1 + r2 * r2 + r3 * r3))
    pv[...] = partial
    pltpu.sync_copy(pv, out_hbm.at[pl.ds(wid * 16, 16)])


def kernel(h, labels, centers):
    B, D = h.shape
    rows_per_w = B // _NW
    lab = jnp.asarray(labels, dtype=jnp.int32).reshape((1,))
    mesh = plsc.VectorSubcoreMesh(core_axis_name="c", subcore_axis_name="s")
    partials = pl.kernel(
        functools.partial(_sc_body, rows_per_w, float(B)),
        out_type=jax.ShapeDtypeStruct((_NW * 16,), jnp.float32),
        mesh=mesh,
        compiler_params=pltpu.CompilerParams(use_tc_tiling_on_sc=False),
        scratch_types=[
            pltpu.VMEM((1,), jnp.int32),
            pltpu.VMEM((rows_per_w, D), jnp.float32),
            pltpu.VMEM((1, D), jnp.float32),
            pltpu.VMEM((16,), jnp.float32),
            pltpu.SemaphoreType.DMA,
            pltpu.SemaphoreType.DMA,
        ],
    )(h, lab, centers)
    return (jnp.sum(partials) / (B * D)).astype(jnp.float32)


# speedup vs baseline: 1.3324x; 1.3324x over previous
import functools

import jax
import jax.numpy as jnp
from jax import lax
from jax.experimental import pallas as pl
from jax.experimental.pallas import tpu as pltpu
from jax.experimental.pallas import tpu_sc as plsc

_NC = 2
_NS = 16
_NW = _NC * _NS


def _sc_body(rows_per_w, batch, h_hbm, lab_hbm, c_hbm, out_hbm,
             idx_v, xv, crow_v, pv, sem, gsem):
    cid = lax.axis_index("c")
    sid = lax.axis_index("s")
    wid = sid * _NC + cid
    base = wid * rows_per_w

    h_cp = pltpu.async_copy(h_hbm.at[pl.ds(base, rows_per_w), :], xv, sem)
    pltpu.sync_copy(lab_hbm, idx_v)
    slab = idx_v[...][0]
    pltpu.async_copy(c_hbm.at[pl.ds(slab, 1), :], crow_v, gsem).wait()
    h_cp.wait()

    zeros = jnp.zeros((16,), jnp.float32)

    def body(r, carry):
        c0, c1, c2, c3, sq = carry
        v0 = xv[r, pl.ds(0, 16)]
        v1 = xv[r, pl.ds(16, 16)]
        v2 = xv[r, pl.ds(32, 16)]
        v3 = xv[r, pl.ds(48, 16)]
        sq = sq + v0 * v0 + v1 * v1 + v2 * v2 + v3 * v3
        return (c0 + v0, c1 + v1, c2 + v2, c3 + v3, sq)

    c0, c1, c2, c3, sq = lax.fori_loop(
        0, rows_per_w, body, (zeros, zeros, zeros, zeros, zeros)
    )

    r0 = crow_v[0, pl.ds(0, 16)]
    r1 = crow_v[0, pl.ds(16, 16)]
    r2 = crow_v[0, pl.ds(32, 16)]
    r3 = crow_v[0, pl.ds(48, 16)]
    csq_share = jnp.float32(batch / _NW)
    partial = (sq - 2.0 * (c0 * r0 + c1 * r1 + c2 * r2 + c3 * r3)
               + csq_share * (r0 * r0 + r1 * r1 + r2 * r2 + r3 * r3))
    pv[...] = partial
    pltpu.sync_copy(pv, out_hbm.at[pl.ds(wid * 16, 16)])


def kernel(h, labels, centers):
    B, D = h.shape
    rows_per_w = B // _NW
    lab = jnp.full((16,), labels, dtype=jnp.int32)
    mesh = plsc.VectorSubcoreMesh(core_axis_name="c", subcore_axis_name="s")
    partials = pl.kernel(
        functools.partial(_sc_body, rows_per_w, float(B)),
        out_type=jax.ShapeDtypeStruct((_NW * 16,), jnp.float32),
        mesh=mesh,
        scratch_types=[
            pltpu.VMEM((16,), jnp.int32),
            pltpu.VMEM((rows_per_w, D), jnp.float32),
            pltpu.VMEM((1, D), jnp.float32),
            pltpu.VMEM((16,), jnp.float32),
            pltpu.SemaphoreType.DMA,
            pltpu.SemaphoreType.DMA,
        ],
    )(h, lab, centers)
    return (jnp.sum(partials) / (B * D)).astype(jnp.float32)
